# baseline (speedup 1.0000x reference)
import functools

import jax
import jax.numpy as jnp
from jax.experimental import pallas as pl
from jax.experimental.pallas import tpu as pltpu

_B = 8
_N = 4096
_M = 4096
_N_T = 512
_K_PAD = 128


def _chamfer_body(xf_ref, ytf_ref, xb_ref, ytb_ref, out_ref, colmin_ref, acc_ref):
    b = pl.program_id(0)
    n = pl.program_id(1)
    last_b = pl.num_programs(0) - 1
    last_n = pl.num_programs(1) - 1

    @pl.when(jnp.logical_and(b == 0, n == 0))
    def _init_acc():
        acc_ref[0] = 0.0

    @pl.when(n == 0)
    def _init_colmin():
        colmin_ref[...] = jnp.full((1, _M), jnp.inf, jnp.float32)

    x = xf_ref[0]
    yt = ytf_ref[0]

    x2 = jnp.sum(x * x, axis=1, keepdims=True)
    y2 = jnp.sum(yt * yt, axis=0, keepdims=True)

    xy = jax.lax.dot_general(
        xb_ref[0],
        ytb_ref[0],
        dimension_numbers=(((1,), (0,)), ((), ())),
        preferred_element_type=jnp.float32,
    )

    e = y2 - 2.0 * xy

    rowmin = jnp.maximum(jnp.min(e, axis=1, keepdims=True) + x2, 0.0)
    acc_ref[0] += jnp.sum(rowmin)

    d = e + x2
    colmin_ref[...] = jnp.minimum(colmin_ref[...], jnp.min(d, axis=0, keepdims=True))

    @pl.when(n == last_n)
    def _flush_colmin():
        acc_ref[0] += jnp.sum(jnp.maximum(colmin_ref[...], 0.0))

    @pl.when(jnp.logical_and(b == last_b, n == last_n))
    def _emit():
        out_ref[0] = acc_ref[0] * (1.0 / (_B * _N))


@functools.partial(jax.jit, static_argnames=())
def kernel(input, target):
    yt = target.transpose(0, 2, 1)
    xb = jnp.pad(input.astype(jnp.bfloat16), ((0, 0), (0, 0), (0, _K_PAD - 3)))
    ytb = jnp.pad(yt.astype(jnp.bfloat16), ((0, 0), (0, _K_PAD - 3), (0, 0)))
    out = pl.pallas_call(
        _chamfer_body,
        grid=(_B, _N // _N_T),
        in_specs=[
            pl.BlockSpec((1, _N_T, 3), lambda b, n: (b, n, 0)),
            pl.BlockSpec((1, 3, _M), lambda b, n: (b, 0, 0)),
            pl.BlockSpec((1, _N_T, _K_PAD), lambda b, n: (b, n, 0)),
            pl.BlockSpec((1, _K_PAD, _M), lambda b, n: (b, 0, 0)),
        ],
        out_specs=pl.BlockSpec((1,), lambda b, n: (0,), memory_space=pltpu.SMEM),
        out_shape=jax.ShapeDtypeStruct((1,), jnp.float32),
        scratch_shapes=[
            pltpu.VMEM((1, _M), jnp.float32),
            pltpu.SMEM((1,), jnp.float32),
        ],
    )(input, yt, xb, ytb)
    return out[0]


# speedup vs baseline: 1.1438x; 1.1438x over previous
import functools

import jax
import jax.numpy as jnp
from jax.experimental import pallas as pl
from jax.experimental.pallas import tpu as pltpu

_B = 8
_N = 4096
_M = 4096
_N_T = 2048
_K_PAD = 16
_M_C = 1024


def _chamfer_body(xa_ref, ya_ref, out_ref, colmin_ref, acc_ref):
    b = pl.program_id(0)
    n = pl.program_id(1)
    last_b = pl.num_programs(0) - 1
    last_n = pl.num_programs(1) - 1

    @pl.when(jnp.logical_and(b == 0, n == 0))
    def _init_acc():
        acc_ref[0] = 0.0

    @pl.when(n == 0)
    def _init_colmin():
        colmin_ref[...] = jnp.full((1, _M), jnp.inf, jnp.float32)

    xa = xa_ref[0]
    rowmin = None
    for c in range(_M // _M_C):
        sl = slice(c * _M_C, (c + 1) * _M_C)
        d = jax.lax.dot_general(
            xa,
            ya_ref[0][:, sl],
            dimension_numbers=(((1,), (0,)), ((), ())),
            preferred_element_type=jnp.float32,
        )
        rm = jnp.min(d, axis=1, keepdims=True)
        rowmin = rm if rowmin is None else jnp.minimum(rowmin, rm)
        colmin_ref[:, sl] = jnp.minimum(
            colmin_ref[:, sl], jnp.min(d, axis=0, keepdims=True)
        )

    acc_ref[0] += jnp.sum(jnp.maximum(rowmin, 0.0))

    @pl.when(n == last_n)
    def _flush_colmin():
        acc_ref[0] += jnp.sum(jnp.maximum(colmin_ref[...], 0.0))

    @pl.when(jnp.logical_and(b == last_b, n == last_n))
    def _emit():
        out_ref[0] = acc_ref[0] * (1.0 / (_B * _N))


def _hi_lo(v):
    hi = v.astype(jnp.bfloat16)
    lo = (v - hi.astype(jnp.float32)).astype(jnp.bfloat16)
    return hi, lo


@functools.partial(jax.jit, static_argnames=())
def kernel(input, target):
    x2hi, x2lo = _hi_lo(jnp.sum(input * input, axis=-1, keepdims=True))
    y2hi, y2lo = _hi_lo(jnp.sum(target * target, axis=-1, keepdims=True))
    ones = jnp.ones((_B, _N, 1), jnp.bfloat16)
    xa = jnp.concatenate(
        [input.astype(jnp.bfloat16), x2hi, x2lo, ones, ones], axis=-1
    )
    ya = jnp.concatenate(
        [(-2.0 * target.astype(jnp.bfloat16).astype(jnp.float32)).astype(jnp.bfloat16),
         ones, ones, y2hi, y2lo],
        axis=-1,
    )
    xa = jnp.pad(xa, ((0, 0), (0, 0), (0, _K_PAD - 7)))
    ya = jnp.pad(ya, ((0, 0), (0, 0), (0, _K_PAD - 7))).transpose(0, 2, 1)

    out = pl.pallas_call(
        _chamfer_body,
        grid=(_B, _N // _N_T),
        in_specs=[
            pl.BlockSpec((1, _N_T, _K_PAD), lambda b, n: (b, n, 0)),
            pl.BlockSpec((1, _K_PAD, _M), lambda b, n: (b, 0, 0)),
        ],
        out_specs=pl.BlockSpec((1,), lambda b, n: (0,), memory_space=pltpu.SMEM),
        out_shape=jax.ShapeDtypeStruct((1,), jnp.float32),
        scratch_shapes=[
            pltpu.VMEM((1, _M), jnp.float32),
            pltpu.SMEM((1,), jnp.float32),
        ],
    )(xa, ya)
    return out[0]


# speedup vs baseline: 1.4385x; 1.2577x over previous
import functools

import jax
import jax.numpy as jnp
from jax.experimental import pallas as pl
from jax.experimental.pallas import tpu as pltpu

_B = 8
_N = 4096
_M = 4096
_N_T = 2048
_K_PAD = 16
_M_C = 1024


def _chamfer_body(xf_ref, ytf_ref, xb_ref, ytb_ref, out_ref, colmin_ref, acc_ref):
    b = pl.program_id(0)
    n = pl.program_id(1)
    last_b = pl.num_programs(0) - 1
    last_n = pl.num_programs(1) - 1

    @pl.when(jnp.logical_and(b == 0, n == 0))
    def _init_acc():
        acc_ref[0] = 0.0

    @pl.when(n == 0)
    def _init_colmin():
        colmin_ref[...] = jnp.full((1, _M), jnp.inf, jnp.float32)

    x = xf_ref[0]
    yt = ytf_ref[0]
    x2 = jnp.sum(x * x, axis=1, keepdims=True)
    y2 = jnp.sum(yt * yt, axis=0, keepdims=True)

    xb = xb_ref[0]
    rowmin_e = None
    for c in range(_M // _M_C):
        sl = slice(c * _M_C, (c + 1) * _M_C)
        xy2 = jax.lax.dot_general(
            xb,
            ytb_ref[0][:, sl],
            dimension_numbers=(((1,), (0,)), ((), ())),
            preferred_element_type=jnp.float32,
        )
        e = y2[:, sl] + xy2
        rm = jnp.min(e, axis=1, keepdims=True)
        rowmin_e = rm if rowmin_e is None else jnp.minimum(rowmin_e, rm)
        colmin_ref[:, sl] = jnp.minimum(
            colmin_ref[:, sl], jnp.min(e + x2, axis=0, keepdims=True)
        )

    acc_ref[0] += jnp.sum(jnp.maximum(rowmin_e + x2, 0.0))

    @pl.when(n == last_n)
    def _flush_colmin():
        acc_ref[0] += jnp.sum(jnp.maximum(colmin_ref[...], 0.0))

    @pl.when(jnp.logical_and(b == last_b, n == last_n))
    def _emit():
        out_ref[0] = acc_ref[0] * (1.0 / (_B * _N))


@functools.partial(jax.jit, static_argnames=())
def kernel(input, target):
    yt = target.transpose(0, 2, 1)
    xb = jnp.pad(input.astype(jnp.bfloat16), ((0, 0), (0, 0), (0, _K_PAD - 3)))
    ytb = jnp.pad(
        (-2.0 * target.astype(jnp.bfloat16).astype(jnp.float32)).astype(jnp.bfloat16),
        ((0, 0), (0, 0), (0, _K_PAD - 3)),
    ).transpose(0, 2, 1)

    out = pl.pallas_call(
        _chamfer_body,
        grid=(_B, _N // _N_T),
        in_specs=[
            pl.BlockSpec((1, _N_T, 3), lambda b, n: (b, n, 0)),
            pl.BlockSpec((1, 3, _M), lambda b, n: (b, 0, 0)),
            pl.BlockSpec((1, _N_T, _K_PAD), lambda b, n: (b, n, 0)),
            pl.BlockSpec((1, _K_PAD, _M), lambda b, n: (b, 0, 0)),
        ],
        out_specs=pl.BlockSpec((1,), lambda b, n: (0,), memory_space=pltpu.SMEM),
        out_shape=jax.ShapeDtypeStruct((1,), jnp.float32),
        scratch_shapes=[
            pltpu.VMEM((1, _M), jnp.float32),
            pltpu.SMEM((1,), jnp.float32),
        ],
    )(input, yt, xb, ytb)
    return out[0]


# speedup vs baseline: 1.4916x; 1.0369x over previous
import functools

import jax
import jax.numpy as jnp
from jax.experimental import pallas as pl
from jax.experimental.pallas import tpu as pltpu

_B = 8
_N = 4096
_M = 4096
_N_T = 4096
_K_PAD = 16
_M_C = 1024


def _chamfer_body(xf_ref, ytf_ref, xb_ref, ytb_ref, out_ref, colmin_ref, acc_ref):
    b = pl.program_id(0)
    n = pl.program_id(1)
    last_b = pl.num_programs(0) - 1
    last_n = pl.num_programs(1) - 1

    @pl.when(jnp.logical_and(b == 0, n == 0))
    def _init_acc():
        acc_ref[0] = 0.0

    @pl.when(n == 0)
    def _init_colmin():
        colmin_ref[...] = jnp.full((1, _M), jnp.inf, jnp.float32)

    x = xf_ref[0]
    yt = ytf_ref[0]
    x2 = jnp.sum(x * x, axis=1, keepdims=True)
    y2 = jnp.sum(yt * yt, axis=0, keepdims=True)

    xb = xb_ref[0]
    rowmin_e = None
    for c in range(_M // _M_C):
        sl = slice(c * _M_C, (c + 1) * _M_C)
        xy2 = jax.lax.dot_general(
            xb,
            ytb_ref[0][:, sl],
            dimension_numbers=(((1,), (0,)), ((), ())),
            preferred_element_type=jnp.float32,
        )
        e = y2[:, sl] + xy2
        rm = jnp.min(e, axis=1, keepdims=True)
        rowmin_e = rm if rowmin_e is None else jnp.minimum(rowmin_e, rm)
        colmin_ref[:, sl] = jnp.minimum(
            colmin_ref[:, sl], jnp.min(e + x2, axis=0, keepdims=True)
        )

    acc_ref[0] += jnp.sum(jnp.maximum(rowmin_e + x2, 0.0))

    @pl.when(n == last_n)
    def _flush_colmin():
        acc_ref[0] += jnp.sum(jnp.maximum(colmin_ref[...], 0.0))

    @pl.when(jnp.logical_and(b == last_b, n == last_n))
    def _emit():
        out_ref[0] = acc_ref[0] * (1.0 / (_B * _N))


@functools.partial(jax.jit, static_argnames=())
def kernel(input, target):
    yt = target.transpose(0, 2, 1)
    xb = jnp.pad(input.astype(jnp.bfloat16), ((0, 0), (0, 0), (0, _K_PAD - 3)))
    ytb = jnp.pad(
        (-2.0 * target.astype(jnp.bfloat16).astype(jnp.float32)).astype(jnp.bfloat16),
        ((0, 0), (0, 0), (0, _K_PAD - 3)),
    ).transpose(0, 2, 1)

    out = pl.pallas_call(
        _chamfer_body,
        grid=(_B, _N // _N_T),
        in_specs=[
            pl.BlockSpec((1, _N_T, 3), lambda b, n: (b, n, 0)),
            pl.BlockSpec((1, 3, _M), lambda b, n: (b, 0, 0)),
            pl.BlockSpec((1, _N_T, _K_PAD), lambda b, n: (b, n, 0)),
            pl.BlockSpec((1, _K_PAD, _M), lambda b, n: (b, 0, 0)),
        ],
        out_specs=pl.BlockSpec((1,), lambda b, n: (0,), memory_space=pltpu.SMEM),
        out_shape=jax.ShapeDtypeStruct((1,), jnp.float32),
        scratch_shapes=[
            pltpu.VMEM((1, _M), jnp.float32),
            pltpu.SMEM((1,), jnp.float32),
        ],
    )(input, yt, xb, ytb)
    return out[0]
